# baseline (speedup 1.0000x reference)
import functools

import jax
import jax.numpy as jnp
from jax import lax
from jax.experimental import pallas as pl
from jax.experimental.pallas import tpu as pltpu
from jax.experimental.pallas import tpu_sc as plsc

B, T, D = 16, 2048, 512
NQ, S = 100, 16
NP = B * NQ * S
LANES = 16
NC, NS = 2, 16
NW = NC * NS
PPW = NP // NW
CHUNK = 40
NCH = PPW // CHUNK
DV = D // LANES


def _tc_prep(spans_ref, idxf_ref, idxc_ref, wb_ref):
    spans = spans_ref[...]
    start = spans[:, 0:1] * (T - 1)
    end = spans[:, 1:2] * (T - 1)
    base = (lax.broadcasted_iota(jnp.int32, (B * NQ, S), 0) // NQ) * T

    steps = lax.broadcasted_iota(jnp.int32, (B * NQ, S), 1).astype(
        jnp.float32) * (1.0 / (S - 1))
    sp = start + steps * (end - start)
    idxf = jnp.clip(sp.astype(jnp.int32), 0, T - 2)
    idxf_ref[...] = idxf + base
    idxc_ref[...] = idxf + base + 1

    s_col = lax.broadcasted_iota(jnp.int32, (B * NQ, S * LANES), 1) // LANES
    steps_b = s_col.astype(jnp.float32) * (1.0 / (S - 1))
    sp_b = start + steps_b * (end - start)
    idxf_b = jnp.clip(sp_b.astype(jnp.int32), 0, T - 2)
    wb_ref[...] = sp_b - idxf_b.astype(jnp.float32)


def _sc_body(table, idx, wb, out,
             idx_v, wb_v, gbuf0, gbuf1, sem0, sem1):
    wid = lax.axis_index("s") * NC + lax.axis_index("c")
    pltpu.sync_copy(idx.at[wid], idx_v)
    pltpu.sync_copy(wb.at[wid], wb_v)
    base_out = wid * PPW
    gbufs = (gbuf0, gbuf1)
    sems = (sem0, sem1)

    pltpu.async_copy(table.at[idx_v.at[0]], gbuf0, sem0)

    def process(j, k):
        jn = jnp.minimum(j + 1, NCH - 1)
        pltpu.async_copy(table.at[idx_v.at[jn]], gbufs[1 - k], sems[1 - k])
        pltpu.make_async_copy(table.at[idx_v.at[j]], gbufs[k], sems[k]).wait()
        buf = gbufs[k]

        def pt_body(p, c2):
            pg = j * CHUNK + p
            wv = wb_v[pg // 8, pl.ds((pg % 8) * LANES, LANES)]
            for d in range(DV):
                sl = pl.ds(d * LANES, LANES)
                f = buf[p, sl]
                c = buf[p + CHUNK, sl]
                buf[p, sl] = f + wv * (c - f)
            return c2

        lax.fori_loop(0, CHUNK, pt_body, 0)
        pltpu.sync_copy(buf.at[pl.ds(0, CHUNK)],
                        out.at[pl.ds(base_out + j * CHUNK, CHUNK)])

    def loop_body(jj, carry):
        process(jj * 2, 0)
        process(jj * 2 + 1, 1)
        return carry

    lax.fori_loop(0, NCH // 2, loop_body, 0)
    pltpu.make_async_copy(table.at[idx_v.at[0]], gbuf0, sem0).wait()


_sc_call = functools.partial(
    pl.kernel,
    mesh=plsc.VectorSubcoreMesh(core_axis_name="c", subcore_axis_name="s"),
    out_type=jax.ShapeDtypeStruct((NP, D), jnp.float32),
    scratch_types=[
        pltpu.VMEM((NCH, 2 * CHUNK), jnp.int32),
        pltpu.VMEM((PPW // 8, 8 * LANES), jnp.float32),
        pltpu.VMEM((2 * CHUNK, D), jnp.float32),
        pltpu.VMEM((2 * CHUNK, D), jnp.float32),
        pltpu.SemaphoreType.DMA,
        pltpu.SemaphoreType.DMA,
    ],
)(_sc_body)


def kernel(video_features, spans):
    table = video_features.reshape(B * T, D)
    idxf, idxc, wb = pl.pallas_call(
        _tc_prep,
        out_shape=[
            jax.ShapeDtypeStruct((B * NQ, S), jnp.int32),
            jax.ShapeDtypeStruct((B * NQ, S), jnp.int32),
            jax.ShapeDtypeStruct((B * NQ, S * LANES), jnp.float32),
        ],
    )(spans.reshape(B * NQ, 2))
    idx = jnp.concatenate(
        [idxf.reshape(NW, NCH, CHUNK), idxc.reshape(NW, NCH, CHUNK)], axis=-1)
    out = _sc_call(table, idx, wb.reshape(NW, PPW // 8, 8 * LANES))
    return out.reshape(B, NQ, S, D)


# speedup vs baseline: 1.4198x; 1.4198x over previous
import functools

import jax
import jax.numpy as jnp
from jax import lax
from jax.experimental import pallas as pl
from jax.experimental.pallas import tpu as pltpu
from jax.experimental.pallas import tpu_sc as plsc

B, T, D = 16, 2048, 512
NQ, S = 100, 16
NP = B * NQ * S
LANES = 16
NC, NS = 2, 16
NW = NC * NS
PPW = NP // NW
CHUNK = 32
NCH = PPW // CHUNK
NSLOT = 3
DV = D // LANES


def _tc_prep(spans_ref, idxf_ref, idxc_ref, wb_ref):
    spans = spans_ref[...]
    start = spans[:, 0:1] * (T - 1)
    end = spans[:, 1:2] * (T - 1)
    base = (lax.broadcasted_iota(jnp.int32, (B * NQ, S), 0) // NQ) * T

    steps = lax.broadcasted_iota(jnp.int32, (B * NQ, S), 1).astype(
        jnp.float32) * (1.0 / (S - 1))
    sp = start + steps * (end - start)
    idxf = jnp.clip(sp.astype(jnp.int32), 0, T - 2)
    idxf_ref[...] = idxf + base
    idxc_ref[...] = idxf + base + 1

    s_col = lax.broadcasted_iota(jnp.int32, (B * NQ, S * LANES), 1) // LANES
    steps_b = s_col.astype(jnp.float32) * (1.0 / (S - 1))
    sp_b = start + steps_b * (end - start)
    idxf_b = jnp.clip(sp_b.astype(jnp.int32), 0, T - 2)
    wb_ref[...] = sp_b - idxf_b.astype(jnp.float32)


def _sc_body(table, idxf, idxc, wb, out,
             idxf_v, idxc_v, wb_v,
             gbuf0, gbuf1, gbuf2, semg0, semg1, semg2, semo0, semo1, semo2):
    wid = lax.axis_index("s") * NC + lax.axis_index("c")
    pltpu.sync_copy(idxf.at[wid], idxf_v)
    pltpu.sync_copy(idxc.at[wid], idxc_v)
    pltpu.sync_copy(wb.at[wid], wb_v)
    base_out = wid * PPW
    gbufs = (gbuf0, gbuf1, gbuf2)
    semg = (semg0, semg1, semg2)
    semo = (semo0, semo1, semo2)

    def fire_gather(j, k):
        pltpu.async_copy(table.at[idxf_v.at[j]],
                         gbufs[k].at[pl.ds(0, CHUNK)], semg[k])
        pltpu.async_copy(table.at[idxc_v.at[j]],
                         gbufs[k].at[pl.ds(CHUNK, CHUNK)], semg[k])

    def wait_gather(j, k):
        pltpu.make_async_copy(table.at[idxf_v.at[j]],
                              gbufs[k].at[pl.ds(0, CHUNK)], semg[k]).wait()
        pltpu.make_async_copy(table.at[idxc_v.at[j]],
                              gbufs[k].at[pl.ds(CHUNK, CHUNK)], semg[k]).wait()

    def fire_writeout(j, k):
        pltpu.async_copy(gbufs[k].at[pl.ds(0, CHUNK)],
                         out.at[pl.ds(base_out + j * CHUNK, CHUNK)], semo[k])

    def wait_writeout(j, k):
        pltpu.make_async_copy(gbufs[k].at[pl.ds(0, CHUNK)],
                              out.at[pl.ds(base_out + j * CHUNK, CHUNK)],
                              semo[k]).wait()

    def blend(j, k):
        buf = gbufs[k]

        def pt_body(p, c2):
            pg = j * CHUNK + p
            wv = wb_v[pg // 8, pl.ds((pg % 8) * LANES, LANES)]
            for d in range(DV):
                sl = pl.ds(d * LANES, LANES)
                f = buf[p, sl]
                c = buf[p + CHUNK, sl]
                buf[p, sl] = f + wv * (c - f)
            return c2

        lax.fori_loop(0, CHUNK, pt_body, 0)

    def steady(j, k, kn):
        wait_writeout(j - 2, kn)
        fire_gather(j + 1, kn)
        wait_gather(j, k)
        blend(j, k)
        fire_writeout(j, k)

    fire_gather(0, 0)
    fire_gather(1, 1)
    wait_gather(0, 0)
    blend(0, 0)
    fire_writeout(0, 0)
    fire_gather(2, 2)
    wait_gather(1, 1)
    blend(1, 1)
    fire_writeout(1, 1)

    def loop_body(g, carry):
        for u in range(NSLOT):
            steady(g * NSLOT + 2 + u, (2 + u) % NSLOT, u)
        return carry

    lax.fori_loop(0, (NCH - 4) // NSLOT, loop_body, 0)
    steady(NCH - 2, (NCH - 2) % NSLOT, (NCH - 1) % NSLOT)

    wait_gather(NCH - 1, (NCH - 1) % NSLOT)
    blend(NCH - 1, (NCH - 1) % NSLOT)
    fire_writeout(NCH - 1, (NCH - 1) % NSLOT)
    wait_writeout(NCH - 3, (NCH - 3) % NSLOT)
    wait_writeout(NCH - 2, (NCH - 2) % NSLOT)
    wait_writeout(NCH - 1, (NCH - 1) % NSLOT)


_sc_call = functools.partial(
    pl.kernel,
    mesh=plsc.VectorSubcoreMesh(core_axis_name="c", subcore_axis_name="s"),
    out_type=jax.ShapeDtypeStruct((NP, D), jnp.float32),
    scratch_types=[
        pltpu.VMEM((NCH, CHUNK), jnp.int32),
        pltpu.VMEM((NCH, CHUNK), jnp.int32),
        pltpu.VMEM((PPW // 8, 8 * LANES), jnp.float32),
        pltpu.VMEM((2 * CHUNK, D), jnp.float32),
        pltpu.VMEM((2 * CHUNK, D), jnp.float32),
        pltpu.VMEM((2 * CHUNK, D), jnp.float32),
        pltpu.SemaphoreType.DMA,
        pltpu.SemaphoreType.DMA,
        pltpu.SemaphoreType.DMA,
        pltpu.SemaphoreType.DMA,
        pltpu.SemaphoreType.DMA,
        pltpu.SemaphoreType.DMA,
    ],
)(_sc_body)


def kernel(video_features, spans):
    table = video_features.reshape(B * T, D)
    idxf, idxc, wb = pl.pallas_call(
        _tc_prep,
        out_shape=[
            jax.ShapeDtypeStruct((B * NQ, S), jnp.int32),
            jax.ShapeDtypeStruct((B * NQ, S), jnp.int32),
            jax.ShapeDtypeStruct((B * NQ, S * LANES), jnp.float32),
        ],
    )(spans.reshape(B * NQ, 2))
    out = _sc_call(table,
                   idxf.reshape(NW, NCH, CHUNK),
                   idxc.reshape(NW, NCH, CHUNK),
                   wb.reshape(NW, PPW // 8, 8 * LANES))
    return out.reshape(B, NQ, S, D)
